# Initial kernel scaffold; baseline (speedup 1.0000x reference)
import jax
import jax.numpy as jnp
from jax.experimental import pallas as pl


def kernel(x, edge_index, batch, params):
    raise NotImplementedError("write your pallas kernel here")


# speedup vs baseline: 3.6549x; 3.6549x over previous
import functools

import jax
import jax.numpy as jnp
from jax import lax
from jax.experimental import pallas as pl
from jax.experimental.pallas import tpu as pltpu
from jax.experimental.pallas import tpu_sc as plsc

N = 10000
E = 320000
D = 128
G = 64

NC = 2
NS = 16
NW = NC * NS
CHUNK = 128

_mesh = plsc.VectorSubcoreMesh(core_axis_name="c", subcore_axis_name="s")


def _pad_to(n, m):
    return ((n + m - 1) // m) * m


def _make_seg_sum(pe, nacc, nout):
    ept = pe // NW
    chunks = ept // CHUNK
    zrows = nacc // NS
    zreps = zrows // 16
    rpt = nout // NS

    @functools.partial(
        pl.kernel,
        mesh=_mesh,
        out_type=jax.ShapeDtypeStruct((NC * nout, D), jnp.float32),
        scratch_types=[
            pltpu.VMEM((CHUNK,), jnp.int32),
            pltpu.VMEM((CHUNK,), jnp.int32),
            pltpu.VMEM((CHUNK, D), jnp.float32),
            pltpu.VMEM((16, D), jnp.float32),
            pltpu.VMEM_SHARED((nacc, D), jnp.float32),
            pltpu.SemaphoreType.DMA,
        ],
    )
    def k(table_hbm, src_hbm, dst_hbm, out_hbm, sidx, didx, rows, zbuf, acc, sem):
        cid = lax.axis_index("c")
        sid = lax.axis_index("s")
        wid = cid * NS + sid
        zero16 = jnp.zeros((16,), jnp.float32)
        for i in range(16):
            for c8 in range(D // 16):
                zbuf[i, pl.ds(c8 * 16, 16)] = zero16

        def zbody(i, _):
            pltpu.sync_copy(zbuf, acc.at[pl.ds(sid * zrows + i * 16, 16)])
            return 0

        lax.fori_loop(0, zreps, zbody, 0)
        plsc.subcore_barrier()

        base = wid * ept

        def body(ci, _):
            off = base + ci * CHUNK
            pltpu.sync_copy(src_hbm.at[pl.ds(off, CHUNK)], sidx)
            pltpu.sync_copy(dst_hbm.at[pl.ds(off, CHUNK)], didx)
            pltpu.async_copy(table_hbm.at[sidx], rows, sem).wait()
            pltpu.sync_copy(rows, acc.at[didx], add=True)
            return 0

        lax.fori_loop(0, chunks, body, 0)
        plsc.subcore_barrier()
        pltpu.sync_copy(
            acc.at[pl.ds(sid * rpt, rpt)],
            out_hbm.at[pl.ds(cid * nout + sid * rpt, rpt)],
        )

    return k


def _make_seg_count(pe, nacc, nout):
    ept = pe // NW
    chunks = ept // CHUNK
    zrows = nacc // NS
    zreps = zrows // 16
    rpt = nout // NS

    @functools.partial(
        pl.kernel,
        mesh=_mesh,
        out_type=jax.ShapeDtypeStruct((NC * nout, D), jnp.float32),
        scratch_types=[
            pltpu.VMEM((CHUNK,), jnp.int32),
            pltpu.VMEM((CHUNK, D), jnp.float32),
            pltpu.VMEM((16, D), jnp.float32),
            pltpu.VMEM_SHARED((nacc, D), jnp.float32),
        ],
    )
    def k(ones_hbm, dst_hbm, out_hbm, didx, ones, zbuf, acc):
        cid = lax.axis_index("c")
        sid = lax.axis_index("s")
        wid = cid * NS + sid
        zero16 = jnp.zeros((16,), jnp.float32)
        pltpu.sync_copy(ones_hbm, ones)
        for i in range(16):
            for c8 in range(D // 16):
                zbuf[i, pl.ds(c8 * 16, 16)] = zero16

        def zbody(i, _):
            pltpu.sync_copy(zbuf, acc.at[pl.ds(sid * zrows + i * 16, 16)])
            return 0

        lax.fori_loop(0, zreps, zbody, 0)
        plsc.subcore_barrier()

        base = wid * ept

        def body(ci, _):
            off = base + ci * CHUNK
            pltpu.sync_copy(dst_hbm.at[pl.ds(off, CHUNK)], didx)
            pltpu.sync_copy(ones, acc.at[didx], add=True)
            return 0

        lax.fori_loop(0, chunks, body, 0)
        plsc.subcore_barrier()
        pltpu.sync_copy(
            acc.at[pl.ds(sid * rpt, rpt)],
            out_hbm.at[pl.ds(cid * nout + sid * rpt, rpt)],
        )

    return k


_PE_EDGE = _pad_to(E, NW * CHUNK)
_PE_NODE = _pad_to(N, NW * CHUNK)
_NOUT_N = _pad_to(N + 1, 128)
_NOUT_G = 128
_NACC_N = _pad_to(_NOUT_N, 256)
_NACC_G = 256

_agg_kernel = _make_seg_sum(_PE_EDGE, _NACC_N, _NOUT_N)
_pool_kernel = _make_seg_sum(_PE_NODE, _NACC_G, _NOUT_G)
_deg_kernel = _make_seg_count(_PE_EDGE, _NACC_N, _NOUT_N)
_cnt_kernel = _make_seg_count(_PE_NODE, _NACC_G, _NOUT_G)


def _layer_body(h_ref, acc_ref, degp_ref, wl_ref, wr_ref, b_ref, g_ref, be_ref,
                o_ref):
    h = h_ref[...]
    acc = acc_ref[0] + acc_ref[1]
    deg = degp_ref[0, :, 0:1] + degp_ref[1, :, 0:1]
    agg = acc / jnp.maximum(deg, 1.0)
    out = (jnp.dot(agg, wl_ref[...], preferred_element_type=jnp.float32)
           + jnp.dot(h, wr_ref[...], preferred_element_type=jnp.float32)
           + b_ref[...])
    mu = jnp.mean(out, axis=0, keepdims=True)
    var = jnp.mean((out - mu) ** 2, axis=0, keepdims=True)
    y = g_ref[...] * (out - mu) * lax.rsqrt(var + 1e-5) + be_ref[...]
    y = jnp.maximum(y, 0.0)
    o_ref[...] = y + h


_layer_tc = pl.pallas_call(
    _layer_body,
    out_shape=jax.ShapeDtypeStruct((N, D), jnp.float32),
)


def _head_body(pool_ref, cntp_ref, m1_ref, c1_ref, m2_ref, c2_ref, m3_ref,
               c3_ref, o_ref):
    s = pool_ref[0] + pool_ref[1]
    cnt = cntp_ref[0, :, 0:1] + cntp_ref[1, :, 0:1]
    pooled = s / jnp.maximum(cnt, 1.0)
    z = jnp.maximum(
        jnp.dot(pooled, m1_ref[...], preferred_element_type=jnp.float32)
        + c1_ref[...], 0.0)
    z = jnp.maximum(
        jnp.dot(z, m2_ref[...], preferred_element_type=jnp.float32)
        + c2_ref[...], 0.0)
    o_ref[...] = (jnp.dot(z, m3_ref[...], preferred_element_type=jnp.float32)
                  + c3_ref[...])


_head_tc = pl.pallas_call(
    _head_body,
    out_shape=jax.ShapeDtypeStruct((G, 8), jnp.float32),
)


def kernel(x, edge_index, batch, params):
    src = edge_index[0]
    dst = edge_index[1]
    i32 = jnp.int32
    src_p = jnp.concatenate([src, jnp.zeros((_PE_EDGE - E,), i32)])
    dst_p = jnp.concatenate([dst, jnp.full((_PE_EDGE - E,), N, i32)])
    node_src = jnp.concatenate(
        [jnp.arange(N, dtype=i32), jnp.zeros((_PE_NODE - N,), i32)])
    node_dst = jnp.concatenate([batch, jnp.full((_PE_NODE - N,), G, i32)])

    ones = jnp.ones((CHUNK, D), jnp.float32)
    degp = _deg_kernel(ones, dst_p).reshape(NC, _NOUT_N, D)[:, :N, :16]
    cntp = _cnt_kernel(ones, node_dst).reshape(NC, _NOUT_G, D)[:, :G, :16]

    h = x
    for i in range(4):
        accs = _agg_kernel(h, src_p, dst_p).reshape(NC, _NOUT_N, D)[:, :N]
        h = _layer_tc(
            h, accs, degp,
            params[f"Wl{i}"], params[f"Wr{i}"],
            params[f"b{i}"].reshape(1, D),
            params[f"g{i}"].reshape(1, D),
            params[f"be{i}"].reshape(1, D),
        )

    pools = _pool_kernel(h, node_src, node_dst).reshape(NC, _NOUT_G, D)[:, :G]
    m3 = jnp.pad(params["M3"], ((0, 0), (0, 7)))
    c3 = jnp.pad(params["c3"], (0, 7)).reshape(1, 8)
    out = _head_tc(
        pools, cntp,
        params["M1"], params["c1"].reshape(1, D),
        params["M2"], params["c2"].reshape(1, D),
        m3, c3,
    )
    return out[:, :1]
